# baseline (speedup 1.0000x reference)
import functools

import jax
import jax.numpy as jnp
from jax import lax
from jax.experimental import pallas as pl
from jax.experimental.pallas import tpu as pltpu
from jax.experimental.pallas import tpu_sc as plsc

B = 16384
F = 26
D = 128
NW = 32
NSPLIT = 4
BP = B // NSPLIT
FR_W = BP // NW
PER_W = FR_W * F
FR_CH = 4
CHF = FR_CH * F
NCH = FR_W // FR_CH
NBUF = 3
STARTS = (0, 16, 32, 48, 64, 80, 88)


def _sc_lookup(feats_hbm, offs_hbm, table_hbm, out_hbm,
               feats_v, idx_v, offs_v, pat_v, rows_v, gsem, ssem):
    wid = lax.axis_index("s") * 2 + lax.axis_index("c")

    pltpu.sync_copy(feats_hbm.at[wid], feats_v)
    pltpu.sync_copy(offs_hbm, offs_v)

    lane = lax.iota(jnp.int32, 16)

    for s in STARTS:
        pat_v[pl.ds(s, 16)] = plsc.load_gather(offs_v, [lax.rem(s + lane, F)])

    def compute_row(j):
        for s in STARTS:
            sl = pl.ds(s, 16)
            feat = plsc.load_gather(feats_v, [j * CHF + s + lane])
            idx_v[j, sl] = feat + pat_v[sl]

    def start_gather(j, slot):
        return pltpu.async_copy(
            table_hbm.at[idx_v.at[j]], rows_v.at[slot], gsem.at[slot])

    def scatter_pair(j, slot):
        return (rows_v.at[slot], out_hbm.at[wid * NCH + j], ssem.at[slot])

    compute_row(0)
    start_gather(0, 0)
    compute_row(1)
    compute_row(2)

    def dma_body(j, _):
        slot = lax.rem(j, NBUF)
        nxt = lax.rem(j + 1, NBUF)

        pltpu.make_async_copy(
            table_hbm.at[idx_v.at[j]], rows_v.at[slot], gsem.at[slot]).wait()
        pltpu.async_copy(*scatter_pair(j, slot))

        @pl.when(j + 1 < NCH)
        def _():
            @pl.when(j + 1 >= NBUF)
            def _():
                pltpu.make_async_copy(*scatter_pair(j + 1 - NBUF, nxt)).wait()
            start_gather(j + 1, nxt)

        @pl.when(j + 3 < NCH)
        def _():
            compute_row(j + 3)
        return 0

    lax.fori_loop(0, NCH, dma_body, 0)

    for jj in range(NCH - NBUF + 1, NCH):
        pltpu.make_async_copy(*scatter_pair(jj, jj % NBUF)).wait()


@jax.jit
def _run(feats_split, offs_pad, table):
    mesh = plsc.VectorSubcoreMesh(core_axis_name="c", subcore_axis_name="s")
    gather_f = functools.partial(
        pl.kernel,
        out_type=jax.ShapeDtypeStruct((NW * NCH, CHF, D), jnp.float32),
        mesh=mesh,
        scratch_types=[
            pltpu.VMEM((PER_W,), jnp.int32),
            pltpu.VMEM((NCH, CHF), jnp.int32),
            pltpu.VMEM((128,), jnp.int32),
            pltpu.VMEM((CHF,), jnp.int32),
            pltpu.VMEM((NBUF, CHF, D), jnp.float32),
            pltpu.SemaphoreType.DMA((NBUF,)),
            pltpu.SemaphoreType.DMA((NBUF,)),
        ],
        compiler_params=pltpu.CompilerParams(needs_layout_passes=False),
    )(_sc_lookup)
    pieces = [
        gather_f(feats_split[p], offs_pad, table).reshape(BP, F, D)
        for p in range(NSPLIT)
    ]
    return jnp.concatenate(pieces, axis=0)


def kernel(features, table, feature_offsets):
    feats_split = features.reshape(NSPLIT, NW, PER_W)
    offs_pad = jnp.pad(feature_offsets, (0, 128 - F))
    return _run(feats_split, offs_pad, table)


# speedup vs baseline: 1.3290x; 1.3290x over previous
import functools

import jax
import jax.numpy as jnp
from jax import lax
from jax.experimental import pallas as pl
from jax.experimental.pallas import tpu as pltpu
from jax.experimental.pallas import tpu_sc as plsc

B = 16384
F = 26
D = 128
NW = 32
FR_W = B // NW
PER_W = FR_W * F
FR_CH = 4
CHF = FR_CH * F
NCH = FR_W // FR_CH
NBUF = 5
PF = 2
STARTS = (0, 16, 32, 48, 64, 80, 88)


def _sc_lookup(feats_hbm, offs_hbm, table_hbm, out_hbm,
               feats_v, idx_v, offs_v, pat_v, rows_v, gsem, ssem):
    wid = lax.axis_index("s") * 2 + lax.axis_index("c")

    pltpu.sync_copy(feats_hbm.at[wid], feats_v)
    pltpu.sync_copy(offs_hbm, offs_v)

    lane = lax.iota(jnp.int32, 16)

    for s in STARTS:
        pat_v[pl.ds(s, 16)] = plsc.load_gather(offs_v, [lax.rem(s + lane, F)])

    def compute_row(j):
        for s in STARTS:
            sl = pl.ds(s, 16)
            feat = plsc.load_gather(feats_v, [j * CHF + s + lane])
            idx_v[j, sl] = feat + pat_v[sl]

    def start_gather(j, slot):
        return pltpu.async_copy(
            table_hbm.at[idx_v.at[j]], rows_v.at[slot], gsem.at[slot])

    def scatter_pair(j, slot):
        return (rows_v.at[slot], out_hbm.at[wid * NCH + j], ssem.at[slot])

    for jj in range(PF):
        compute_row(jj)
        start_gather(jj, jj)
    compute_row(PF)
    compute_row(PF + 1)

    def dma_body(j, _):
        slot = lax.rem(j, NBUF)
        pfs = lax.rem(j + PF, NBUF)

        pltpu.make_async_copy(
            table_hbm.at[idx_v.at[j]], rows_v.at[slot], gsem.at[slot]).wait()
        pltpu.async_copy(*scatter_pair(j, slot))

        @pl.when(j + PF < NCH)
        def _():
            @pl.when(j + PF >= NBUF)
            def _():
                pltpu.make_async_copy(*scatter_pair(j + PF - NBUF, pfs)).wait()
            start_gather(j + PF, pfs)

        @pl.when(j + PF + 2 < NCH)
        def _():
            compute_row(j + PF + 2)
        return 0

    lax.fori_loop(0, NCH, dma_body, 0)

    for jj in range(NCH - NBUF + PF, NCH):
        pltpu.make_async_copy(*scatter_pair(jj, jj % NBUF)).wait()


@jax.jit
def _run(feats_split, offs_pad, table):
    mesh = plsc.VectorSubcoreMesh(core_axis_name="c", subcore_axis_name="s")
    gather_f = functools.partial(
        pl.kernel,
        out_type=jax.ShapeDtypeStruct((NW * NCH, CHF, D), jnp.float32),
        mesh=mesh,
        scratch_types=[
            pltpu.VMEM((PER_W,), jnp.int32),
            pltpu.VMEM((NCH, CHF), jnp.int32),
            pltpu.VMEM((128,), jnp.int32),
            pltpu.VMEM((CHF,), jnp.int32),
            pltpu.VMEM((NBUF, CHF, D), jnp.float32),
            pltpu.SemaphoreType.DMA((NBUF,)),
            pltpu.SemaphoreType.DMA((NBUF,)),
        ],
        compiler_params=pltpu.CompilerParams(needs_layout_passes=False),
    )(_sc_lookup)
    return gather_f(feats_split, offs_pad, table).reshape(B, F, D)


def kernel(features, table, feature_offsets):
    feats_split = features.reshape(NW, PER_W)
    offs_pad = jnp.pad(feature_offsets, (0, 128 - F))
    return _run(feats_split, offs_pad, table)


# speedup vs baseline: 2.2189x; 1.6696x over previous
import functools

import jax
import jax.numpy as jnp
from jax import lax
from jax.experimental import pallas as pl
from jax.experimental.pallas import tpu as pltpu
from jax.experimental.pallas import tpu_sc as plsc

B = 16384
F = 26
D = 128
NW = 32
FR_W = B // NW
PER_W = FR_W * F
FR_CH = 4
CHF = FR_CH * F
NCH = FR_W // FR_CH
NBUF = 5
PF = 2
STARTS = (0, 16, 32, 48, 64, 80, 88)


def _sc_lookup(feats_hbm, offs_hbm, table_hbm, out_hbm,
               feats_v, idx_v, offs_v, pat_v, rows_v, gsem, ssem):
    wid = lax.axis_index("s") * 2 + lax.axis_index("c")
    b0 = wid * FR_W

    pltpu.sync_copy(feats_hbm.at[wid], feats_v)
    pltpu.sync_copy(offs_hbm, offs_v)

    lane = lax.iota(jnp.int32, 16)

    for s in STARTS:
        pat_v[pl.ds(s, 16)] = plsc.load_gather(offs_v, [lax.rem(s + lane, F)])

    def compute_row(j):
        for s in STARTS:
            sl = pl.ds(s, 16)
            feat = plsc.load_gather(feats_v, [j * CHF + s + lane])
            idx_v[j, sl] = feat + pat_v[sl]

    def start_gather(j, slot):
        return pltpu.async_copy(
            table_hbm.at[idx_v.at[j]], rows_v.at[slot], gsem.at[slot])

    def scatter_pairs(j, slot):
        return [(rows_v.at[slot, pl.ds(k * F, F)],
                 out_hbm.at[(b0 + j * FR_CH) + k], ssem.at[slot])
                for k in range(FR_CH)]

    for jj in range(PF):
        compute_row(jj)
        start_gather(jj, jj)
    compute_row(PF)
    compute_row(PF + 1)

    def dma_body(j, _):
        slot = lax.rem(j, NBUF)
        pfs = lax.rem(j + PF, NBUF)

        pltpu.make_async_copy(
            table_hbm.at[idx_v.at[j]], rows_v.at[slot], gsem.at[slot]).wait()
        for p in scatter_pairs(j, slot):
            pltpu.async_copy(*p)

        @pl.when(j + PF < NCH)
        def _():
            @pl.when(j + PF >= NBUF)
            def _():
                for p in scatter_pairs(j + PF - NBUF, pfs):
                    pltpu.make_async_copy(*p).wait()
            start_gather(j + PF, pfs)

        @pl.when(j + PF + 2 < NCH)
        def _():
            compute_row(j + PF + 2)
        return 0

    lax.fori_loop(0, NCH, dma_body, 0)

    for jj in range(NCH - NBUF + PF, NCH):
        for p in scatter_pairs(jj, jj % NBUF):
            pltpu.make_async_copy(*p).wait()


@jax.jit
def _run(feats_flat, offs_pad, table):
    mesh = plsc.VectorSubcoreMesh(core_axis_name="c", subcore_axis_name="s")
    gather_f = functools.partial(
        pl.kernel,
        out_type=jax.ShapeDtypeStruct((B, F, D), jnp.float32),
        mesh=mesh,
        scratch_types=[
            pltpu.VMEM((PER_W,), jnp.int32),
            pltpu.VMEM((NCH, CHF), jnp.int32),
            pltpu.VMEM((128,), jnp.int32),
            pltpu.VMEM((CHF,), jnp.int32),
            pltpu.VMEM((NBUF, CHF, D), jnp.float32),
            pltpu.SemaphoreType.DMA((NBUF,)),
            pltpu.SemaphoreType.DMA((NBUF,)),
        ],
        compiler_params=pltpu.CompilerParams(
            needs_layout_passes=False, use_tc_tiling_on_sc=True),
    )(_sc_lookup)
    return gather_f(feats_flat, offs_pad, table)


def kernel(features, table, feature_offsets):
    feats_flat = features.reshape(NW, PER_W)
    offs_pad = jnp.pad(feature_offsets, (0, 128 - F))
    return _run(feats_flat, offs_pad, table)


# speedup vs baseline: 2.2371x; 1.0082x over previous
import functools

import jax
import jax.numpy as jnp
from jax import lax
from jax.experimental import pallas as pl
from jax.experimental.pallas import tpu as pltpu
from jax.experimental.pallas import tpu_sc as plsc

B = 16384
F = 26
D = 128
NW = 32
FR_W = B // NW
PER_W = FR_W * F
FR_CH = 4
CHF = FR_CH * F
NCH = FR_W // FR_CH
NBUF = 6
PF = 3
STARTS = (0, 16, 32, 48, 64, 80, 88)


def _sc_lookup(feats_hbm, offs_hbm, table_hbm, out_hbm,
               feats_v, idx_v, offs_v, pat_v, rows_v, gsem, ssem):
    wid = lax.axis_index("s") * 2 + lax.axis_index("c")
    b0 = wid * FR_W

    pltpu.sync_copy(feats_hbm.at[wid], feats_v)
    pltpu.sync_copy(offs_hbm, offs_v)

    lane = lax.iota(jnp.int32, 16)

    for s in STARTS:
        pat_v[pl.ds(s, 16)] = plsc.load_gather(offs_v, [lax.rem(s + lane, F)])

    def compute_row(j):
        for s in STARTS:
            sl = pl.ds(s, 16)
            feat = plsc.load_gather(feats_v, [j * CHF + s + lane])
            idx_v[j, sl] = feat + pat_v[sl]

    def start_gather(j, slot):
        return pltpu.async_copy(
            table_hbm.at[idx_v.at[j]], rows_v.at[slot], gsem.at[slot])

    def scatter_pairs(j, slot):
        return [(rows_v.at[slot, pl.ds(k * F, F)],
                 out_hbm.at[(b0 + j * FR_CH) + k], ssem.at[slot])
                for k in range(FR_CH)]

    for jj in range(PF):
        compute_row(jj)
        start_gather(jj, jj)
    compute_row(PF)
    compute_row(PF + 1)

    def dma_body(j, _):
        slot = lax.rem(j, NBUF)
        pfs = lax.rem(j + PF, NBUF)

        pltpu.make_async_copy(
            table_hbm.at[idx_v.at[j]], rows_v.at[slot], gsem.at[slot]).wait()
        for p in scatter_pairs(j, slot):
            pltpu.async_copy(*p)

        @pl.when(j + PF < NCH)
        def _():
            @pl.when(j + PF >= NBUF)
            def _():
                for p in scatter_pairs(j + PF - NBUF, pfs):
                    pltpu.make_async_copy(*p).wait()
            start_gather(j + PF, pfs)

        @pl.when(j + PF + 2 < NCH)
        def _():
            compute_row(j + PF + 2)
        return 0

    lax.fori_loop(0, NCH, dma_body, 0)

    for jj in range(NCH - NBUF + PF, NCH):
        for p in scatter_pairs(jj, jj % NBUF):
            pltpu.make_async_copy(*p).wait()


@jax.jit
def _run(feats_flat, offs_pad, table):
    mesh = plsc.VectorSubcoreMesh(core_axis_name="c", subcore_axis_name="s")
    gather_f = functools.partial(
        pl.kernel,
        out_type=jax.ShapeDtypeStruct((B, F, D), jnp.float32),
        mesh=mesh,
        scratch_types=[
            pltpu.VMEM((PER_W,), jnp.int32),
            pltpu.VMEM((NCH, CHF), jnp.int32),
            pltpu.VMEM((128,), jnp.int32),
            pltpu.VMEM((CHF,), jnp.int32),
            pltpu.VMEM((NBUF, CHF, D), jnp.float32),
            pltpu.SemaphoreType.DMA((NBUF,)),
            pltpu.SemaphoreType.DMA((NBUF,)),
        ],
        compiler_params=pltpu.CompilerParams(
            needs_layout_passes=False, use_tc_tiling_on_sc=True),
    )(_sc_lookup)
    return gather_f(feats_flat, offs_pad, table)


def kernel(features, table, feature_offsets):
    feats_flat = features.reshape(NW, PER_W)
    offs_pad = jnp.pad(feature_offsets, (0, 128 - F))
    return _run(feats_flat, offs_pad, table)
